# baseline (speedup 1.0000x reference)
import jax
import jax.numpy as jnp
from jax import lax
from jax.experimental import pallas as pl
from jax.experimental.pallas import tpu as pltpu
from jax.experimental.pallas import tpu_sc as plsc

_N = 100000
_VOCAB = 94
_DIM = 128
_C = 128
_NCHUNK = -(-_N // _C)
_NBUF = 4
_L = 16
_GROUPS = _C // _L

_info = plsc.get_sparse_core_info()
_NCORES = _info.num_cores
_NSUB = _info.num_subcores
_NW = _NCORES * _NSUB
_MAXT = -(-_NCHUNK // _NW)
_TLOOP = -(-_MAXT // _NBUF) * _NBUF


def _body(idx_hbm, table_hbm, out_hbm, table_v, idx_v, rows_v, *sems):
    wid = lax.axis_index("s") * _NCORES + lax.axis_index("c")
    isems = list(sems[:_NBUF])
    wsems = list(sems[_NBUF:])

    pltpu.sync_copy(table_hbm, table_v)

    def base_of(t):
        cid = wid + t * _NW
        return lax.min(cid * _C, _N - _C)

    def active(t):
        return (wid + t * _NW) < _NCHUNK

    def start_idx(t, b):
        pltpu.async_copy(idx_hbm.at[pl.ds(base_of(t), _C)], idx_v.at[b], isems[b])

    def gather_chunk(b):
        @plsc.parallel_loop(0, _GROUPS)
        def _(g):
            idx16 = idx_v[b, pl.ds(g * _L, _L)]
            for l in range(_L):
                base = idx16[l] << 7
                r = g * _L + l
                for c in range(_DIM // _L):
                    rows_v[b, r, pl.ds(16 * c, _L)] = table_v[pl.ds(base + 16 * c, _L)]

    for b in range(_NBUF):
        start_idx(b, b)

    @pl.loop(0, _TLOOP, step=_NBUF)
    def _(g):
        for b in range(_NBUF):
            t = g + b

            @pl.when(active(t))
            def _():
                pltpu.make_async_copy(
                    idx_hbm.at[pl.ds(base_of(t), _C)], idx_v.at[b], isems[b]
                ).wait()

            @pl.when(active(t) & (t >= _NBUF))
            def _():
                pltpu.make_async_copy(
                    rows_v.at[b], out_hbm.at[pl.ds(base_of(t - _NBUF), _C)], wsems[b]
                ).wait()

            @pl.when(active(t))
            def _():
                gather_chunk(b)
                pltpu.async_copy(
                    rows_v.at[b], out_hbm.at[pl.ds(base_of(t), _C)], wsems[b]
                )

            @pl.when(active(t + _NBUF))
            def _():
                start_idx(t + _NBUF, b)

    for t in range(_MAXT - _NBUF - 1, _MAXT):
        b = t % _NBUF

        @pl.when(active(t) & ~active(t + _NBUF))
        def _():
            pltpu.make_async_copy(
                rows_v.at[b], out_hbm.at[pl.ds(base_of(t), _C)], wsems[b]
            ).wait()


def kernel(atomic_numbers, embedding_weight):
    idx = atomic_numbers.astype(jnp.int32)
    table_flat = embedding_weight.reshape(_VOCAB * _DIM)
    run = pl.kernel(
        _body,
        out_type=jax.ShapeDtypeStruct((_N, _DIM), jnp.float32),
        mesh=plsc.VectorSubcoreMesh(core_axis_name="c", subcore_axis_name="s"),
        scratch_types=[
            pltpu.VMEM((_VOCAB * _DIM,), jnp.float32),
            pltpu.VMEM((_NBUF, _C), jnp.int32),
            pltpu.VMEM((_NBUF, _C, _DIM), jnp.float32),
        ]
        + [pltpu.SemaphoreType.DMA] * (2 * _NBUF),
    )
    return run(idx, table_flat)


# speedup vs baseline: 1.9887x; 1.9887x over previous
import jax
import jax.numpy as jnp
from jax import lax
from jax.experimental import pallas as pl
from jax.experimental.pallas import tpu as pltpu
from jax.experimental.pallas import tpu_sc as plsc

_N = 100000
_VOCAB = 94
_DIM = 128
_C = 128
_NCHUNK = -(-_N // _C)
_NBUF = 3

_info = plsc.get_sparse_core_info()
_NCORES = _info.num_cores
_NSUB = _info.num_subcores
_NW = _NCORES * _NSUB
_MAXT = -(-_NCHUNK // _NW)
_TLOOP = -(-_MAXT // _NBUF) * _NBUF


def _body(idx_hbm, table_hbm, out_hbm, table_sh, idx_v, rows_v, *sems):
    wid = lax.axis_index("s") * _NCORES + lax.axis_index("c")
    isems = list(sems[:_NBUF])
    gsems = list(sems[_NBUF:2 * _NBUF])
    wsems = list(sems[2 * _NBUF:])

    @pl.when(lax.axis_index("s") == 0)
    def _():
        pltpu.sync_copy(table_hbm, table_sh)

    plsc.subcore_barrier()

    def base_of(t):
        cid = wid + t * _NW
        return lax.min(cid * _C, _N - _C)

    def active(t):
        return (wid + t * _NW) < _NCHUNK

    def start_idx(t, b):
        pltpu.async_copy(idx_hbm.at[pl.ds(base_of(t), _C)], idx_v.at[b], isems[b])

    for b in range(_NBUF):
        start_idx(b, b)

    @pl.loop(0, _TLOOP, step=_NBUF)
    def _(g):
        for b in range(_NBUF):
            t = g + b

            @pl.when(active(t))
            def _():
                pltpu.make_async_copy(
                    idx_hbm.at[pl.ds(base_of(t), _C)], idx_v.at[b], isems[b]
                ).wait()

            @pl.when(active(t) & (t >= _NBUF))
            def _():
                pltpu.make_async_copy(
                    rows_v.at[b], out_hbm.at[pl.ds(base_of(t - _NBUF), _C)], wsems[b]
                ).wait()

            @pl.when(active(t))
            def _():
                pltpu.async_copy(
                    table_sh.at[idx_v.at[b]], rows_v.at[b], gsems[b]
                ).wait()
                pltpu.async_copy(
                    rows_v.at[b], out_hbm.at[pl.ds(base_of(t), _C)], wsems[b]
                )

            @pl.when(active(t + _NBUF))
            def _():
                start_idx(t + _NBUF, b)

    for t in range(_MAXT - _NBUF - 1, _MAXT):
        b = t % _NBUF

        @pl.when(active(t) & ~active(t + _NBUF))
        def _():
            pltpu.make_async_copy(
                rows_v.at[b], out_hbm.at[pl.ds(base_of(t), _C)], wsems[b]
            ).wait()


def kernel(atomic_numbers, embedding_weight):
    idx = atomic_numbers.astype(jnp.int32)
    run = pl.kernel(
        _body,
        out_type=jax.ShapeDtypeStruct((_N, _DIM), jnp.float32),
        mesh=plsc.VectorSubcoreMesh(core_axis_name="c", subcore_axis_name="s"),
        scratch_types=[
            pltpu.VMEM_SHARED((_VOCAB, _DIM), jnp.float32),
            pltpu.VMEM((_NBUF, _C), jnp.int32),
            pltpu.VMEM((_NBUF, _C, _DIM), jnp.float32),
        ]
        + [pltpu.SemaphoreType.DMA] * (3 * _NBUF),
    )
    return run(idx, embedding_weight)
